# Initial kernel scaffold; baseline (speedup 1.0000x reference)
import functools

import jax
import jax.numpy as jnp
from jax import lax
from jax.experimental import pallas as pl
from jax.experimental.pallas import tpu as pltpu
from jax.experimental.pallas import tpu_sc as plsc

N = 10000
E = 320000
D = 128

NCORE = 2
NSUB = 16
NW = NCORE * NSUB
CK = 128
CH = 80
EP = NW * CH * CK
ZR = 10016
RPT = ZR // NSUB

_mesh = plsc.VectorSubcoreMesh(core_axis_name="c", subcore_axis_name="s")


def _deg_kernel(dstp, zeros16, ones_rows):
  @functools.partial(
      pl.kernel,
      out_type=jax.ShapeDtypeStruct((NCORE, ZR, 16), jnp.float32),
      mesh=_mesh,
      scratch_types=[
          pltpu.VMEM((CH, CK), jnp.int32),
          pltpu.VMEM((CK, 16), jnp.float32),
          pltpu.VMEM_SHARED((ZR, 16), jnp.float32),
      ],
  )
  def deg_k(dst_hbm, zero_hbm, ones_hbm, out_hbm, dstv, onesv, dsh):
    c = lax.axis_index("c")
    s = lax.axis_index("s")
    wid = s * NCORE + c
    base = s * RPT
    pltpu.sync_copy(zero_hbm.at[pl.ds(base, RPT)], dsh.at[pl.ds(base, RPT)])
    pltpu.sync_copy(dst_hbm.at[wid], dstv)
    pltpu.sync_copy(ones_hbm, onesv)
    plsc.subcore_barrier()

    @pl.loop(0, CH)
    def _(j):
      pltpu.sync_copy(onesv, dsh.at[dstv.at[j]], add=True)

    plsc.subcore_barrier()
    pltpu.sync_copy(dsh.at[pl.ds(base, RPT)], out_hbm.at[c, pl.ds(base, RPT)])

  return deg_k(dstp, zeros16, ones_rows)


def _agg(srcp, dstp, y, zeros, d):
  @functools.partial(
      pl.kernel,
      out_type=jax.ShapeDtypeStruct((NCORE, ZR, d), jnp.float32),
      mesh=_mesh,
      scratch_types=[
          pltpu.VMEM((CH, CK), jnp.int32),
          pltpu.VMEM((CH, CK), jnp.int32),
          pltpu.VMEM((CK, d), jnp.float32),
          pltpu.VMEM_SHARED((ZR, d), jnp.float32),
          pltpu.SemaphoreType.DMA,
      ],
  )
  def agg_k(src_hbm, dst_hbm, y_hbm, zero_hbm, out_hbm, srcv, dstv, buf, zsh,
            sem):
    c = lax.axis_index("c")
    s = lax.axis_index("s")
    wid = s * NCORE + c
    base = s * RPT

    @pl.when(c == 0)
    def _():
      pltpu.sync_copy(y_hbm.at[pl.ds(base, RPT)], zsh.at[pl.ds(base, RPT)])

    @pl.when(c == 1)
    def _():
      pltpu.sync_copy(zero_hbm.at[pl.ds(base, RPT)], zsh.at[pl.ds(base, RPT)])

    pltpu.sync_copy(src_hbm.at[wid], srcv)
    pltpu.sync_copy(dst_hbm.at[wid], dstv)
    plsc.subcore_barrier()

    @pl.loop(0, CH)
    def _(j):
      pltpu.async_copy(y_hbm.at[srcv.at[j]], buf, sem).wait()
      pltpu.sync_copy(buf, zsh.at[dstv.at[j]], add=True)

    plsc.subcore_barrier()
    pltpu.sync_copy(zsh.at[pl.ds(base, RPT)], out_hbm.at[c, pl.ds(base, RPT)])

  return agg_k(srcp, dstp, y, zeros)


def _tc_prep(degp, nodes_p):
  def body(deg_ref, x_ref, dinv_ref, y_ref):
    deg = deg_ref[0, :, 0:1] + deg_ref[1, :, 0:1] + 1.0
    dinv = lax.rsqrt(deg)
    dinv_ref[...] = dinv
    y_ref[...] = x_ref[...] * dinv

  return pl.pallas_call(
      body,
      out_shape=(
          jax.ShapeDtypeStruct((ZR, 1), jnp.float32),
          jax.ShapeDtypeStruct((ZR, D), jnp.float32),
      ),
  )(degp, nodes_p)


def _tc_mid(z1, dinv, W1, b1, W2p):
  def body(z_ref, dinv_ref, w1_ref, b1_ref, w2_ref, y2_ref):
    dinv = dinv_ref[...]
    zs = (z_ref[0] + z_ref[1]) * dinv
    h = jnp.dot(zs, w1_ref[...], preferred_element_type=jnp.float32)
    h = jnp.maximum(h + b1_ref[...], 0.0)
    y2 = jnp.dot(h, w2_ref[...], preferred_element_type=jnp.float32)
    y2_ref[...] = y2 * dinv

  return pl.pallas_call(
      body,
      out_shape=jax.ShapeDtypeStruct((ZR, 160), jnp.float32),
  )(z1, dinv, W1, b1.reshape(1, -1), W2p)


def _tc_tail(z2, dinv, b2p, W3p, b3, W4, b4):
  def body(z_ref, dinv_ref, b2_ref, w3_ref, b3_ref, w4_ref, b4_ref, o_ref):
    dinv = dinv_ref[...]
    h2 = jnp.maximum((z_ref[0] + z_ref[1]) * dinv + b2_ref[...], 0.0)
    h3 = jnp.dot(h2, w3_ref[...], preferred_element_type=jnp.float32)
    h3 = jnp.maximum(h3 + b3_ref[...], 0.0)
    o = jnp.dot(h3, w4_ref[...], preferred_element_type=jnp.float32)
    o_ref[...] = jax.nn.sigmoid(o + b4_ref[...])

  return pl.pallas_call(
      body,
      out_shape=jax.ShapeDtypeStruct((ZR, 1), jnp.float32),
  )(z2, dinv, b2p.reshape(1, -1), W3p, b3.reshape(1, -1), W4, b4.reshape(1, -1))


def kernel(nodes, edges, W1, b1, W2, b2, W3, b3, W4, b4):
  f32 = jnp.float32
  nodes_p = jnp.zeros((ZR, D), f32).at[:N].set(nodes)
  pad = EP - E
  srcp = jnp.concatenate(
      [edges[0], jnp.zeros((pad,), jnp.int32)]).reshape(NW, CH, CK)
  dstp = jnp.concatenate(
      [edges[1], jnp.full((pad,), N, jnp.int32)]).reshape(NW, CH, CK)

  degp = _deg_kernel(dstp, jnp.zeros((ZR, 16), f32), jnp.ones((CK, 16), f32))
  dinv, y1 = _tc_prep(degp, nodes_p)
  z1 = _agg(srcp, dstp, y1, jnp.zeros((ZR, D), f32), D)

  W2p = jnp.zeros((200, 160), f32).at[:, :150].set(W2)
  y2 = _tc_mid(z1, dinv, W1, b1, W2p)
  z2 = _agg(srcp, dstp, y2, jnp.zeros((ZR, 160), f32), 160)

  b2p = jnp.zeros((160,), f32).at[:150].set(b2)
  W3p = jnp.zeros((160, 100), f32).at[:150].set(W3)
  out = _tc_tail(z2, dinv, b2p, W3p, b3, W4, b4)
  return out[:N]


# speedup vs baseline: 10.3064x; 10.3064x over previous
import functools

import jax
import jax.numpy as jnp
from jax import lax
from jax.experimental import pallas as pl
from jax.experimental.pallas import tpu as pltpu
from jax.experimental.pallas import tpu_sc as plsc

N = 10000
E = 320000
D = 128

NCORE = 2
NSUB = 16
NW = NCORE * NSUB
CK = 128
CH = 80
CH2 = 160
EP = NW * CH * CK
ZR = 10112
RPT = ZR // NSUB

_mesh = plsc.VectorSubcoreMesh(core_axis_name="c", subcore_axis_name="s")
_sc_params = pltpu.CompilerParams(use_tc_tiling_on_sc=False)


def _deg_kernel(dstp, zeros16, ones_rows):
  @functools.partial(
      pl.kernel,
      out_type=jax.ShapeDtypeStruct((NCORE, ZR, 16), jnp.float32),
      mesh=_mesh,
      compiler_params=_sc_params,
      scratch_types=[
          pltpu.VMEM((CH, CK), jnp.int32),
          pltpu.VMEM((CK, 16), jnp.float32),
          pltpu.VMEM_SHARED((ZR, 16), jnp.float32),
      ],
  )
  def deg_k(dst_hbm, zero_hbm, ones_hbm, out_hbm, dstv, onesv, dsh):
    c = lax.axis_index("c")
    s = lax.axis_index("s")
    wid = s * NCORE + c
    base = s * RPT
    pltpu.sync_copy(zero_hbm.at[pl.ds(base, RPT)], dsh.at[pl.ds(base, RPT)])
    pltpu.sync_copy(dst_hbm.at[wid], dstv)
    pltpu.sync_copy(ones_hbm, onesv)
    plsc.subcore_barrier()

    @pl.loop(0, CH)
    def _(j):
      pltpu.sync_copy(onesv, dsh.at[dstv.at[j]], add=True)

    plsc.subcore_barrier()
    pltpu.sync_copy(dsh.at[pl.ds(base, RPT)], out_hbm.at[c, pl.ds(base, RPT)])

  return deg_k(dstp, zeros16, ones_rows)


def _agg(srcp, dstp, ya, yb, d):
  hd = d // 2

  @functools.partial(
      pl.kernel,
      out_type=jax.ShapeDtypeStruct((ZR, d), jnp.float32),
      mesh=_mesh,
      compiler_params=_sc_params,
      scratch_types=[
          pltpu.VMEM((CH2, CK), jnp.int32),
          pltpu.VMEM((CH2, CK), jnp.int32),
          pltpu.VMEM((CK, hd), jnp.float32),
          pltpu.VMEM_SHARED((ZR, hd), jnp.float32),
          pltpu.SemaphoreType.DMA,
      ],
  )
  def agg_k(src_hbm, dst_hbm, ya_hbm, yb_hbm, out_hbm, srcv, dstv, buf, zsh,
            sem):
    c = lax.axis_index("c")
    s = lax.axis_index("s")
    base = s * RPT

    @pl.when(c == 0)
    def _():
      pltpu.sync_copy(ya_hbm.at[pl.ds(base, RPT)], zsh.at[pl.ds(base, RPT)])

    @pl.when(c == 1)
    def _():
      pltpu.sync_copy(yb_hbm.at[pl.ds(base, RPT)], zsh.at[pl.ds(base, RPT)])

    pltpu.sync_copy(src_hbm.at[s], srcv)
    pltpu.sync_copy(dst_hbm.at[s], dstv)
    plsc.subcore_barrier()

    @pl.loop(0, CH2)
    def _(j):
      @pl.when(c == 0)
      def _():
        pltpu.async_copy(ya_hbm.at[srcv.at[j]], buf, sem).wait()

      @pl.when(c == 1)
      def _():
        pltpu.async_copy(yb_hbm.at[srcv.at[j]], buf, sem).wait()

      pltpu.sync_copy(buf, zsh.at[dstv.at[j]], add=True)

    plsc.subcore_barrier()
    pltpu.sync_copy(zsh.at[pl.ds(base, RPT)],
                    out_hbm.at[pl.ds(base, RPT), pl.ds(c * hd, hd)])

  return agg_k(srcp, dstp, ya, yb)


def _tc_prep(degp, nodes_p):
  def body(deg_ref, x_ref, dinv_ref, ya_ref, yb_ref):
    deg = deg_ref[0, :, 0:1] + deg_ref[1, :, 0:1] + 1.0
    dinv = lax.rsqrt(deg)
    dinv_ref[...] = dinv
    y = x_ref[...] * dinv
    ya_ref[...] = y[:, :D // 2]
    yb_ref[...] = y[:, D // 2:]

  return pl.pallas_call(
      body,
      out_shape=(
          jax.ShapeDtypeStruct((ZR, 1), jnp.float32),
          jax.ShapeDtypeStruct((ZR, D // 2), jnp.float32),
          jax.ShapeDtypeStruct((ZR, D // 2), jnp.float32),
      ),
  )(degp, nodes_p)


def _tc_mid(z1, dinv, W1, b1, W2p):
  def body(z_ref, dinv_ref, w1_ref, b1_ref, w2_ref, ya_ref, yb_ref):
    dinv = dinv_ref[...]
    zs = z_ref[...] * dinv
    h = jnp.dot(zs, w1_ref[...], preferred_element_type=jnp.float32)
    h = jnp.maximum(h + b1_ref[...], 0.0)
    y2 = jnp.dot(h, w2_ref[...], preferred_element_type=jnp.float32)
    y2 = y2 * dinv
    ya_ref[...] = y2[:, :80]
    yb_ref[...] = y2[:, 80:]

  return pl.pallas_call(
      body,
      out_shape=(
          jax.ShapeDtypeStruct((ZR, 80), jnp.float32),
          jax.ShapeDtypeStruct((ZR, 80), jnp.float32),
      ),
  )(z1, dinv, W1, b1.reshape(1, -1), W2p)


def _tc_tail(z2, dinv, b2p, W3p, b3, W4, b4):
  def body(z_ref, dinv_ref, b2_ref, w3_ref, b3_ref, w4_ref, b4_ref, o_ref):
    dinv = dinv_ref[...]
    h2 = jnp.maximum(z_ref[...] * dinv + b2_ref[...], 0.0)
    h3 = jnp.dot(h2, w3_ref[...], preferred_element_type=jnp.float32)
    h3 = jnp.maximum(h3 + b3_ref[...], 0.0)
    o = jnp.dot(h3, w4_ref[...], preferred_element_type=jnp.float32)
    o_ref[...] = jax.nn.sigmoid(o + b4_ref[...])

  return pl.pallas_call(
      body,
      out_shape=jax.ShapeDtypeStruct((ZR, 1), jnp.float32),
  )(z2, dinv, b2p.reshape(1, -1), W3p, b3.reshape(1, -1), W4, b4.reshape(1, -1))


def kernel(nodes, edges, W1, b1, W2, b2, W3, b3, W4, b4):
  f32 = jnp.float32
  nodes_p = jnp.zeros((ZR, D), f32).at[:N].set(nodes)
  pad = EP - E
  src_flat = jnp.concatenate([edges[0], jnp.zeros((pad,), jnp.int32)])
  dst_flat = jnp.concatenate([edges[1], jnp.full((pad,), N, jnp.int32)])

  degp = _deg_kernel(dst_flat.reshape(NW, CH, CK),
                     jnp.zeros((ZR, 16), f32), jnp.ones((CK, 16), f32))
  dinv, y1a, y1b = _tc_prep(degp, nodes_p)

  srcp = src_flat.reshape(NSUB, CH2, CK)
  dstp = dst_flat.reshape(NSUB, CH2, CK)
  z1 = _agg(srcp, dstp, y1a, y1b, D)

  W2p = jnp.zeros((200, 160), f32).at[:, :150].set(W2)
  y2a, y2b = _tc_mid(z1, dinv, W1, b1, W2p)
  z2 = _agg(srcp, dstp, y2a, y2b, 160)

  b2p = jnp.zeros((160,), f32).at[:150].set(b2)
  W3p = jnp.zeros((160, 100), f32).at[:150].set(W3)
  out = _tc_tail(z2, dinv, b2p, W3p, b3, W4, b4)
  return out[:N]


# speedup vs baseline: 12.6813x; 1.2304x over previous
import functools

import jax
import jax.numpy as jnp
from jax import lax
from jax.experimental import pallas as pl
from jax.experimental.pallas import tpu as pltpu
from jax.experimental.pallas import tpu_sc as plsc

N = 10000
E = 320000
D = 128

NCORE = 2
NSUB = 16
NW = NCORE * NSUB
CK = 128
CH = 80
CH2 = 160
EP = NW * CH * CK
ZR = 10112
RPT = ZR // NSUB

_mesh = plsc.VectorSubcoreMesh(core_axis_name="c", subcore_axis_name="s")
_sc_params = pltpu.CompilerParams(use_tc_tiling_on_sc=False)


def _deg_kernel(dstp, zeros16, ones_rows):
  @functools.partial(
      pl.kernel,
      out_type=jax.ShapeDtypeStruct((NCORE, ZR, 16), jnp.float32),
      mesh=_mesh,
      compiler_params=_sc_params,
      scratch_types=[
          pltpu.VMEM((CH, CK), jnp.int32),
          pltpu.VMEM((CK, 16), jnp.float32),
          pltpu.VMEM_SHARED((ZR, 16), jnp.float32),
      ],
  )
  def deg_k(dst_hbm, zero_hbm, ones_hbm, out_hbm, dstv, onesv, dsh):
    c = lax.axis_index("c")
    s = lax.axis_index("s")
    wid = s * NCORE + c
    base = s * RPT
    pltpu.sync_copy(zero_hbm.at[pl.ds(base, RPT)], dsh.at[pl.ds(base, RPT)])
    pltpu.sync_copy(dst_hbm.at[wid], dstv)
    pltpu.sync_copy(ones_hbm, onesv)
    plsc.subcore_barrier()

    @pl.loop(0, CH)
    def _(j):
      pltpu.sync_copy(onesv, dsh.at[dstv.at[j]], add=True)

    plsc.subcore_barrier()
    pltpu.sync_copy(dsh.at[pl.ds(base, RPT)], out_hbm.at[c, pl.ds(base, RPT)])

  return deg_k(dstp, zeros16, ones_rows)


def _agg(srcp, dstp, ya, yb, d):
  hd = d // 2
  K = 4 if hd <= 64 else 2
  M = CH2 // (2 * K)

  @functools.partial(
      pl.kernel,
      out_type=jax.ShapeDtypeStruct((ZR, d), jnp.float32),
      mesh=_mesh,
      compiler_params=_sc_params,
      scratch_types=[
          pltpu.VMEM((K, CK), jnp.int32),
          pltpu.VMEM((K, CK), jnp.int32),
          pltpu.VMEM((K, CK), jnp.int32),
          pltpu.VMEM((K, CK), jnp.int32),
          pltpu.VMEM((K * CK, hd), jnp.float32),
          pltpu.VMEM((K * CK, hd), jnp.float32),
          pltpu.VMEM_SHARED((ZR, hd), jnp.float32),
          pltpu.SemaphoreType.DMA,
          pltpu.SemaphoreType.DMA,
          pltpu.SemaphoreType.DMA,
          pltpu.SemaphoreType.DMA,
      ],
  )
  def agg_k(src_hbm, dst_hbm, ya_hbm, yb_hbm, out_hbm, isrca, isrcb, idsta,
            idstb, bufa, bufb, zsh, gsema, gsemb, ssema, ssemb):
    c = lax.axis_index("c")
    s = lax.axis_index("s")
    base = s * RPT

    @pl.when(c == 0)
    def _():
      pltpu.sync_copy(ya_hbm.at[pl.ds(base, RPT)], zsh.at[pl.ds(base, RPT)])

    @pl.when(c == 1)
    def _():
      pltpu.sync_copy(yb_hbm.at[pl.ds(base, RPT)], zsh.at[pl.ds(base, RPT)])

    plsc.subcore_barrier()

    def core_half(y_hbm):
      def load_idx(j0, isrc, idst):
        pltpu.sync_copy(src_hbm.at[s, pl.ds(j0, K)], isrc)
        pltpu.sync_copy(dst_hbm.at[s, pl.ds(j0, K)], idst)

      def gather_super(isrc, buf, gsem):
        for k in range(K):
          pltpu.async_copy(y_hbm.at[isrc.at[k]],
                           buf.at[pl.ds(k * CK, CK)], gsem)

      def drain_gather(isrc, buf, gsem):
        for k in range(K):
          pltpu.make_async_copy(y_hbm.at[isrc.at[k]],
                                buf.at[pl.ds(k * CK, CK)], gsem).wait()

      def scatter_super(idst, buf, ssem):
        for k in range(K):
          pltpu.async_copy(buf.at[pl.ds(k * CK, CK)],
                           zsh.at[idst.at[k]], ssem, add=True)

      def drain_scatter(idst, buf, ssem):
        for k in range(K):
          pltpu.make_async_copy(buf.at[pl.ds(k * CK, CK)],
                                zsh.at[idst.at[k]], ssem).wait()

      load_idx(0, isrca, idsta)
      gather_super(isrca, bufa, gsema)
      load_idx(K, isrcb, idstb)
      gather_super(isrcb, bufb, gsemb)

      @pl.loop(0, M)
      def _(m):
        j0 = m * 2 * K
        drain_gather(isrca, bufa, gsema)
        scatter_super(idsta, bufa, ssema)
        drain_gather(isrcb, bufb, gsemb)
        scatter_super(idstb, bufb, ssemb)

        @pl.when(m + 1 < M)
        def _():
          drain_scatter(idsta, bufa, ssema)
          load_idx(j0 + 2 * K, isrca, idsta)
          gather_super(isrca, bufa, gsema)
          drain_scatter(idstb, bufb, ssemb)
          load_idx(j0 + 3 * K, isrcb, idstb)
          gather_super(isrcb, bufb, gsemb)

      drain_scatter(idsta, bufa, ssema)
      drain_scatter(idstb, bufb, ssemb)

    @pl.when(c == 0)
    def _():
      core_half(ya_hbm)

    @pl.when(c == 1)
    def _():
      core_half(yb_hbm)

    plsc.subcore_barrier()
    pltpu.sync_copy(zsh.at[pl.ds(base, RPT)],
                    out_hbm.at[pl.ds(base, RPT), pl.ds(c * hd, hd)])

  return agg_k(srcp, dstp, ya, yb)


def _tc_prep(degp, nodes_p):
  def body(deg_ref, x_ref, dinv_ref, ya_ref, yb_ref):
    deg = deg_ref[0, :, 0:1] + deg_ref[1, :, 0:1] + 1.0
    dinv = lax.rsqrt(deg)
    dinv_ref[...] = dinv
    y = x_ref[...] * dinv
    ya_ref[...] = y[:, :D // 2]
    yb_ref[...] = y[:, D // 2:]

  return pl.pallas_call(
      body,
      out_shape=(
          jax.ShapeDtypeStruct((ZR, 1), jnp.float32),
          jax.ShapeDtypeStruct((ZR, D // 2), jnp.float32),
          jax.ShapeDtypeStruct((ZR, D // 2), jnp.float32),
      ),
  )(degp, nodes_p)


def _tc_mid(z1, dinv, W1, b1, W2p):
  def body(z_ref, dinv_ref, w1_ref, b1_ref, w2_ref, ya_ref, yb_ref):
    dinv = dinv_ref[...]
    zs = z_ref[...] * dinv
    h = jnp.dot(zs, w1_ref[...], preferred_element_type=jnp.float32)
    h = jnp.maximum(h + b1_ref[...], 0.0)
    y2 = jnp.dot(h, w2_ref[...], preferred_element_type=jnp.float32)
    y2 = y2 * dinv
    ya_ref[...] = y2[:, :80]
    yb_ref[...] = y2[:, 80:]

  return pl.pallas_call(
      body,
      out_shape=(
          jax.ShapeDtypeStruct((ZR, 80), jnp.float32),
          jax.ShapeDtypeStruct((ZR, 80), jnp.float32),
      ),
  )(z1, dinv, W1, b1.reshape(1, -1), W2p)


def _tc_tail(z2, dinv, b2p, W3p, b3, W4, b4):
  def body(z_ref, dinv_ref, b2_ref, w3_ref, b3_ref, w4_ref, b4_ref, o_ref):
    dinv = dinv_ref[...]
    h2 = jnp.maximum(z_ref[...] * dinv + b2_ref[...], 0.0)
    h3 = jnp.dot(h2, w3_ref[...], preferred_element_type=jnp.float32)
    h3 = jnp.maximum(h3 + b3_ref[...], 0.0)
    o = jnp.dot(h3, w4_ref[...], preferred_element_type=jnp.float32)
    o_ref[...] = jax.nn.sigmoid(o + b4_ref[...])

  return pl.pallas_call(
      body,
      out_shape=jax.ShapeDtypeStruct((ZR, 1), jnp.float32),
  )(z2, dinv, b2p.reshape(1, -1), W3p, b3.reshape(1, -1), W4, b4.reshape(1, -1))


def kernel(nodes, edges, W1, b1, W2, b2, W3, b3, W4, b4):
  f32 = jnp.float32
  nodes_p = jnp.zeros((ZR, D), f32).at[:N].set(nodes)
  pad = EP - E
  src_flat = jnp.concatenate([edges[0], jnp.zeros((pad,), jnp.int32)])
  dst_flat = jnp.concatenate([edges[1], jnp.full((pad,), N, jnp.int32)])

  degp = _deg_kernel(dst_flat.reshape(NW, CH, CK),
                     jnp.zeros((ZR, 16), f32), jnp.ones((CK, 16), f32))
  dinv, y1a, y1b = _tc_prep(degp, nodes_p)

  srcp = src_flat.reshape(NSUB, CH2, CK)
  dstp = dst_flat.reshape(NSUB, CH2, CK)
  z1 = _agg(srcp, dstp, y1a, y1b, D)

  W2p = jnp.zeros((200, 160), f32).at[:, :150].set(W2)
  y2a, y2b = _tc_mid(z1, dinv, W1, b1, W2p)
  z2 = _agg(srcp, dstp, y2a, y2b, 160)

  b2p = jnp.zeros((160,), f32).at[:150].set(b2)
  W3p = jnp.zeros((160, 100), f32).at[:150].set(W3)
  out = _tc_tail(z2, dinv, b2p, W3p, b3, W4, b4)
  return out[:N]


# speedup vs baseline: 13.6801x; 1.0788x over previous
import functools

import jax
import jax.numpy as jnp
from jax import lax
from jax.experimental import pallas as pl
from jax.experimental.pallas import tpu as pltpu
from jax.experimental.pallas import tpu_sc as plsc

N = 10000
E = 320000
D = 128

NCORE = 2
NSUB = 16
NW = NCORE * NSUB
CK = 128
CH = 80
CH2 = 160
EP = NW * CH * CK
ZR = 10112
RPT = ZR // NSUB

_mesh = plsc.VectorSubcoreMesh(core_axis_name="c", subcore_axis_name="s")
_sc_params = pltpu.CompilerParams(use_tc_tiling_on_sc=False)


def _deg_kernel(dstp, zeros16, ones_rows):
  @functools.partial(
      pl.kernel,
      out_type=jax.ShapeDtypeStruct((NCORE, ZR, 16), jnp.float32),
      mesh=_mesh,
      compiler_params=_sc_params,
      scratch_types=[
          pltpu.VMEM((CH, CK), jnp.int32),
          pltpu.VMEM((CK, 16), jnp.float32),
          pltpu.VMEM_SHARED((ZR, 16), jnp.float32),
      ],
  )
  def deg_k(dst_hbm, zero_hbm, ones_hbm, out_hbm, dstv, onesv, dsh):
    c = lax.axis_index("c")
    s = lax.axis_index("s")
    wid = s * NCORE + c
    base = s * RPT
    pltpu.sync_copy(zero_hbm.at[pl.ds(base, RPT)], dsh.at[pl.ds(base, RPT)])
    pltpu.sync_copy(dst_hbm.at[wid], dstv)
    pltpu.sync_copy(ones_hbm, onesv)
    plsc.subcore_barrier()

    @pl.loop(0, CH)
    def _(j):
      pltpu.sync_copy(onesv, dsh.at[dstv.at[j]], add=True)

    plsc.subcore_barrier()
    pltpu.sync_copy(dsh.at[pl.ds(base, RPT)], out_hbm.at[c, pl.ds(base, RPT)])

  return deg_k(dstp, zeros16, ones_rows)


def _agg(srcp, dstp, ya, yb, d):
  hd = d // 2
  K = 2 if hd <= 64 else 1
  NG = CH2 // K
  NB = 4

  @functools.partial(
      pl.kernel,
      out_type=jax.ShapeDtypeStruct((ZR, d), jnp.float32),
      mesh=_mesh,
      compiler_params=_sc_params,
      scratch_types=(
          [pltpu.VMEM((CH2, CK), jnp.int32)]
          + [pltpu.VMEM((K, CK), jnp.int32)] * NB
          + [pltpu.VMEM((K * CK, hd), jnp.float32)] * NB
          + [pltpu.VMEM_SHARED((ZR, hd), jnp.float32)]
          + [pltpu.SemaphoreType.DMA] * (2 * NB)
      ),
  )
  def agg_k(src_hbm, dst_hbm, ya_hbm, yb_hbm, out_hbm, srcv, di0, di1, di2,
            di3, b0, b1, b2, b3, zsh, gs0, gs1, gs2, gs3, ss0, ss1, ss2, ss3):
    dis = [di0, di1, di2, di3]
    bufs = [b0, b1, b2, b3]
    gsems = [gs0, gs1, gs2, gs3]
    ssems = [ss0, ss1, ss2, ss3]
    c = lax.axis_index("c")
    s = lax.axis_index("s")
    base = s * RPT

    @pl.when(c == 0)
    def _():
      pltpu.sync_copy(ya_hbm.at[pl.ds(base, RPT)], zsh.at[pl.ds(base, RPT)])

    @pl.when(c == 1)
    def _():
      pltpu.sync_copy(yb_hbm.at[pl.ds(base, RPT)], zsh.at[pl.ds(base, RPT)])

    pltpu.sync_copy(src_hbm.at[s], srcv)
    plsc.subcore_barrier()

    def core_half(y_hbm):
      def load_idx(g, slot):
        pltpu.sync_copy(dst_hbm.at[s, pl.ds(g * K, K)], dis[slot])

      def gather(g, slot):
        for k in range(K):
          pltpu.async_copy(y_hbm.at[srcv.at[g * K + k]],
                           bufs[slot].at[pl.ds(k * CK, CK)], gsems[slot])

      def drain_gather(g, slot):
        for k in range(K):
          pltpu.make_async_copy(y_hbm.at[srcv.at[g * K + k]],
                                bufs[slot].at[pl.ds(k * CK, CK)],
                                gsems[slot]).wait()

      def scatter(slot):
        for k in range(K):
          pltpu.async_copy(bufs[slot].at[pl.ds(k * CK, CK)],
                           zsh.at[dis[slot].at[k]], ssems[slot], add=True)

      def drain_scatter(slot):
        for k in range(K):
          pltpu.make_async_copy(bufs[slot].at[pl.ds(k * CK, CK)],
                                zsh.at[dis[slot].at[k]], ssems[slot]).wait()

      load_idx(0, 0)
      gather(0, 0)
      load_idx(1, 1)
      gather(1, 1)

      @pl.loop(0, NG // NB)
      def _(t):
        for p in range(NB):
          g = t * NB + p
          pslot = (p + 2) % NB

          @pl.when(g >= 2)
          def _():
            drain_scatter(pslot)

          @pl.when(g + 2 < NG)
          def _():
            load_idx(g + 2, pslot)
            gather(g + 2, pslot)

          drain_gather(g, p)
          scatter(p)

      drain_scatter((NG - 2) % NB)
      drain_scatter((NG - 1) % NB)

    @pl.when(c == 0)
    def _():
      core_half(ya_hbm)

    @pl.when(c == 1)
    def _():
      core_half(yb_hbm)

    plsc.subcore_barrier()
    pltpu.sync_copy(zsh.at[pl.ds(base, RPT)],
                    out_hbm.at[pl.ds(base, RPT), pl.ds(c * hd, hd)])

  return agg_k(srcp, dstp, ya, yb)


def _tc_prep(degp, nodes_p):
  def body(deg_ref, x_ref, dinv_ref, ya_ref, yb_ref):
    deg = deg_ref[0, :, 0:1] + deg_ref[1, :, 0:1] + 1.0
    dinv = lax.rsqrt(deg)
    dinv_ref[...] = dinv
    y = x_ref[...] * dinv
    ya_ref[...] = y[:, :D // 2]
    yb_ref[...] = y[:, D // 2:]

  return pl.pallas_call(
      body,
      out_shape=(
          jax.ShapeDtypeStruct((ZR, 1), jnp.float32),
          jax.ShapeDtypeStruct((ZR, D // 2), jnp.float32),
          jax.ShapeDtypeStruct((ZR, D // 2), jnp.float32),
      ),
  )(degp, nodes_p)


def _tc_mid(z1, dinv, W1, b1, W2p):
  def body(z_ref, dinv_ref, w1_ref, b1_ref, w2_ref, ya_ref, yb_ref):
    dinv = dinv_ref[...]
    zs = z_ref[...] * dinv
    h = jnp.dot(zs, w1_ref[...], preferred_element_type=jnp.float32)
    h = jnp.maximum(h + b1_ref[...], 0.0)
    y2 = jnp.dot(h, w2_ref[...], preferred_element_type=jnp.float32)
    y2 = y2 * dinv
    ya_ref[...] = y2[:, :80]
    yb_ref[...] = y2[:, 80:]

  return pl.pallas_call(
      body,
      out_shape=(
          jax.ShapeDtypeStruct((ZR, 80), jnp.float32),
          jax.ShapeDtypeStruct((ZR, 80), jnp.float32),
      ),
  )(z1, dinv, W1, b1.reshape(1, -1), W2p)


def _tc_tail(z2, dinv, b2p, W3p, b3, W4, b4):
  def body(z_ref, dinv_ref, b2_ref, w3_ref, b3_ref, w4_ref, b4_ref, o_ref):
    dinv = dinv_ref[...]
    h2 = jnp.maximum(z_ref[...] * dinv + b2_ref[...], 0.0)
    h3 = jnp.dot(h2, w3_ref[...], preferred_element_type=jnp.float32)
    h3 = jnp.maximum(h3 + b3_ref[...], 0.0)
    o = jnp.dot(h3, w4_ref[...], preferred_element_type=jnp.float32)
    o_ref[...] = jax.nn.sigmoid(o + b4_ref[...])

  return pl.pallas_call(
      body,
      out_shape=jax.ShapeDtypeStruct((ZR, 1), jnp.float32),
  )(z2, dinv, b2p.reshape(1, -1), W3p, b3.reshape(1, -1), W4, b4.reshape(1, -1))


def kernel(nodes, edges, W1, b1, W2, b2, W3, b3, W4, b4):
  f32 = jnp.float32
  nodes_p = jnp.zeros((ZR, D), f32).at[:N].set(nodes)
  pad = EP - E
  src_flat = jnp.concatenate([edges[0], jnp.zeros((pad,), jnp.int32)])
  dst_flat = jnp.concatenate([edges[1], jnp.full((pad,), N, jnp.int32)])

  degp = _deg_kernel(dst_flat.reshape(NW, CH, CK),
                     jnp.zeros((ZR, 16), f32), jnp.ones((CK, 16), f32))
  dinv, y1a, y1b = _tc_prep(degp, nodes_p)

  srcp = src_flat.reshape(NSUB, CH2, CK)
  dstp = dst_flat.reshape(NSUB, CH2, CK)
  z1 = _agg(srcp, dstp, y1a, y1b, D)

  W2p = jnp.zeros((200, 160), f32).at[:, :150].set(W2)
  y2a, y2b = _tc_mid(z1, dinv, W1, b1, W2p)
  z2 = _agg(srcp, dstp, y2a, y2b, 160)

  b2p = jnp.zeros((160,), f32).at[:150].set(b2)
  W3p = jnp.zeros((160, 100), f32).at[:150].set(W3)
  out = _tc_tail(z2, dinv, b2p, W3p, b3, W4, b4)
  return out[:N]


# speedup vs baseline: 13.9557x; 1.0202x over previous
import functools

import jax
import jax.numpy as jnp
from jax import lax
from jax.experimental import pallas as pl
from jax.experimental.pallas import tpu as pltpu
from jax.experimental.pallas import tpu_sc as plsc

N = 10000
E = 320000
D = 128

NCORE = 2
NSUB = 16
NW = NCORE * NSUB
CK = 128
CH = 80
CH2 = 160
EP = NW * CH * CK
ZR = 10112
RPT = ZR // NSUB

_mesh = plsc.VectorSubcoreMesh(core_axis_name="c", subcore_axis_name="s")
_sc_params = pltpu.CompilerParams(use_tc_tiling_on_sc=False)


def _deg_kernel(dstp, zeros16, ones_rows):
  @functools.partial(
      pl.kernel,
      out_type=jax.ShapeDtypeStruct((NCORE, ZR, 16), jnp.float32),
      mesh=_mesh,
      compiler_params=_sc_params,
      scratch_types=[
          pltpu.VMEM((CH, CK), jnp.int32),
          pltpu.VMEM((CK, 16), jnp.float32),
          pltpu.VMEM_SHARED((ZR, 16), jnp.float32),
      ],
  )
  def deg_k(dst_hbm, zero_hbm, ones_hbm, out_hbm, dstv, onesv, dsh):
    c = lax.axis_index("c")
    s = lax.axis_index("s")
    wid = s * NCORE + c
    base = s * RPT
    pltpu.sync_copy(zero_hbm.at[pl.ds(base, RPT)], dsh.at[pl.ds(base, RPT)])
    pltpu.sync_copy(dst_hbm.at[wid], dstv)
    pltpu.sync_copy(ones_hbm, onesv)
    plsc.subcore_barrier()

    @pl.loop(0, CH)
    def _(j):
      pltpu.sync_copy(onesv, dsh.at[dstv.at[j]], add=True)

    plsc.subcore_barrier()
    pltpu.sync_copy(dsh.at[pl.ds(base, RPT)], out_hbm.at[c, pl.ds(base, RPT)])

  return deg_k(dstp, zeros16, ones_rows)


def _agg(srcp, dstp, ya, yb, d):
  hd = d // 2
  K = 2 if hd <= 64 else 1
  NG = CH2 // K
  NB = 4

  @functools.partial(
      pl.kernel,
      out_type=jax.ShapeDtypeStruct((ZR, d), jnp.float32),
      mesh=_mesh,
      compiler_params=_sc_params,
      scratch_types=(
          [pltpu.VMEM((CH2, CK), jnp.int32)]
          + [pltpu.VMEM((K, CK), jnp.int32)] * NB
          + [pltpu.VMEM((K * CK, hd), jnp.float32)] * NB
          + [pltpu.VMEM_SHARED((ZR, hd), jnp.float32)]
          + [pltpu.SemaphoreType.DMA] * (3 * NB)
      ),
  )
  def agg_k(src_hbm, dst_hbm, ya_hbm, yb_hbm, out_hbm, srcv, di0, di1, di2,
            di3, b0, b1, b2, b3, zsh, gs0, gs1, gs2, gs3, ss0, ss1, ss2, ss3,
            is0, is1, is2, is3):
    dis = [di0, di1, di2, di3]
    bufs = [b0, b1, b2, b3]
    gsems = [gs0, gs1, gs2, gs3]
    ssems = [ss0, ss1, ss2, ss3]
    isems = [is0, is1, is2, is3]
    c = lax.axis_index("c")
    s = lax.axis_index("s")
    base = s * RPT

    @pl.when(c == 0)
    def _():
      pltpu.sync_copy(ya_hbm.at[pl.ds(base, RPT)], zsh.at[pl.ds(base, RPT)])

    @pl.when(c == 1)
    def _():
      pltpu.sync_copy(yb_hbm.at[pl.ds(base, RPT)], zsh.at[pl.ds(base, RPT)])

    pltpu.sync_copy(src_hbm.at[s], srcv)
    plsc.subcore_barrier()

    def core_half(y_hbm):
      def load_idx(g, slot):
        pltpu.async_copy(dst_hbm.at[s, pl.ds(g * K, K)], dis[slot],
                         isems[slot])

      def wait_idx(g, slot):
        pltpu.make_async_copy(dst_hbm.at[s, pl.ds(g * K, K)], dis[slot],
                              isems[slot]).wait()

      def gather(g, slot):
        for k in range(K):
          pltpu.async_copy(y_hbm.at[srcv.at[g * K + k]],
                           bufs[slot].at[pl.ds(k * CK, CK)], gsems[slot])

      def drain_gather(g, slot):
        for k in range(K):
          pltpu.make_async_copy(y_hbm.at[srcv.at[g * K + k]],
                                bufs[slot].at[pl.ds(k * CK, CK)],
                                gsems[slot]).wait()

      def scatter(slot):
        for k in range(K):
          pltpu.async_copy(bufs[slot].at[pl.ds(k * CK, CK)],
                           zsh.at[dis[slot].at[k]], ssems[slot], add=True)

      def drain_scatter(slot):
        for k in range(K):
          pltpu.make_async_copy(bufs[slot].at[pl.ds(k * CK, CK)],
                                zsh.at[dis[slot].at[k]], ssems[slot]).wait()

      for g0 in range(NB - 1):
        load_idx(g0, g0)
        gather(g0, g0)

      @pl.loop(0, NG // NB)
      def _(t):
        for p in range(NB):
          g = t * NB + p
          pslot = (p + NB - 1) % NB

          @pl.when(g >= 1)
          def _():
            drain_scatter(pslot)

          @pl.when(g + NB - 1 < NG)
          def _():
            load_idx(g + NB - 1, pslot)
            gather(g + NB - 1, pslot)

          drain_gather(g, p)
          wait_idx(g, p)
          scatter(p)

      drain_scatter((NG - 1) % NB)

    @pl.when(c == 0)
    def _():
      core_half(ya_hbm)

    @pl.when(c == 1)
    def _():
      core_half(yb_hbm)

    plsc.subcore_barrier()
    pltpu.sync_copy(zsh.at[pl.ds(base, RPT)],
                    out_hbm.at[pl.ds(base, RPT), pl.ds(c * hd, hd)])

  return agg_k(srcp, dstp, ya, yb)


def _tc_prep(degp, nodes_p):
  def body(deg_ref, x_ref, dinv_ref, ya_ref, yb_ref):
    deg = deg_ref[0, :, 0:1] + deg_ref[1, :, 0:1] + 1.0
    dinv = lax.rsqrt(deg)
    dinv_ref[...] = dinv
    y = x_ref[...] * dinv
    ya_ref[...] = y[:, :D // 2]
    yb_ref[...] = y[:, D // 2:]

  return pl.pallas_call(
      body,
      out_shape=(
          jax.ShapeDtypeStruct((ZR, 1), jnp.float32),
          jax.ShapeDtypeStruct((ZR, D // 2), jnp.float32),
          jax.ShapeDtypeStruct((ZR, D // 2), jnp.float32),
      ),
  )(degp, nodes_p)


def _tc_mid(z1, dinv, W1, b1, W2p):
  def body(z_ref, dinv_ref, w1_ref, b1_ref, w2_ref, ya_ref, yb_ref):
    dinv = dinv_ref[...]
    zs = z_ref[...] * dinv
    h = jnp.dot(zs, w1_ref[...], preferred_element_type=jnp.float32)
    h = jnp.maximum(h + b1_ref[...], 0.0)
    y2 = jnp.dot(h, w2_ref[...], preferred_element_type=jnp.float32)
    y2 = y2 * dinv
    ya_ref[...] = y2[:, :80]
    yb_ref[...] = y2[:, 80:]

  return pl.pallas_call(
      body,
      out_shape=(
          jax.ShapeDtypeStruct((ZR, 80), jnp.float32),
          jax.ShapeDtypeStruct((ZR, 80), jnp.float32),
      ),
  )(z1, dinv, W1, b1.reshape(1, -1), W2p)


def _tc_tail(z2, dinv, b2p, W3p, b3, W4, b4):
  def body(z_ref, dinv_ref, b2_ref, w3_ref, b3_ref, w4_ref, b4_ref, o_ref):
    dinv = dinv_ref[...]
    h2 = jnp.maximum(z_ref[...] * dinv + b2_ref[...], 0.0)
    h3 = jnp.dot(h2, w3_ref[...], preferred_element_type=jnp.float32)
    h3 = jnp.maximum(h3 + b3_ref[...], 0.0)
    o = jnp.dot(h3, w4_ref[...], preferred_element_type=jnp.float32)
    o_ref[...] = jax.nn.sigmoid(o + b4_ref[...])

  return pl.pallas_call(
      body,
      out_shape=jax.ShapeDtypeStruct((ZR, 1), jnp.float32),
  )(z2, dinv, b2p.reshape(1, -1), W3p, b3.reshape(1, -1), W4, b4.reshape(1, -1))


def kernel(nodes, edges, W1, b1, W2, b2, W3, b3, W4, b4):
  f32 = jnp.float32
  nodes_p = jnp.zeros((ZR, D), f32).at[:N].set(nodes)
  pad = EP - E
  src_flat = jnp.concatenate([edges[0], jnp.zeros((pad,), jnp.int32)])
  dst_flat = jnp.concatenate([edges[1], jnp.full((pad,), N, jnp.int32)])

  degp = _deg_kernel(dst_flat.reshape(NW, CH, CK),
                     jnp.zeros((ZR, 16), f32), jnp.ones((CK, 16), f32))
  dinv, y1a, y1b = _tc_prep(degp, nodes_p)

  srcp = src_flat.reshape(NSUB, CH2, CK)
  dstp = dst_flat.reshape(NSUB, CH2, CK)
  z1 = _agg(srcp, dstp, y1a, y1b, D)

  W2p = jnp.zeros((200, 160), f32).at[:, :150].set(W2)
  y2a, y2b = _tc_mid(z1, dinv, W1, b1, W2p)
  z2 = _agg(srcp, dstp, y2a, y2b, 160)

  b2p = jnp.zeros((160,), f32).at[:150].set(b2)
  W3p = jnp.zeros((160, 100), f32).at[:150].set(W3)
  out = _tc_tail(z2, dinv, b2p, W3p, b3, W4, b4)
  return out[:N]


# speedup vs baseline: 25.3399x; 1.8157x over previous
import functools

import jax
import jax.numpy as jnp
from jax import lax
from jax.experimental import pallas as pl
from jax.experimental.pallas import tpu as pltpu
from jax.experimental.pallas import tpu_sc as plsc

N = 10000
E = 320000
D = 128

NCORE = 2
NSUB = 16
NW = NCORE * NSUB
CK = 128
CH = 80
CH2 = 160
EP = NW * CH * CK
ZR = 10112
RPT = ZR // NSUB

_mesh = plsc.VectorSubcoreMesh(core_axis_name="c", subcore_axis_name="s")
_sc_params = pltpu.CompilerParams(use_tc_tiling_on_sc=False)


def _deg_kernel(dstp, zeros16, ones_rows):
  @functools.partial(
      pl.kernel,
      out_type=jax.ShapeDtypeStruct((NCORE, ZR, 16), jnp.float32),
      mesh=_mesh,
      compiler_params=_sc_params,
      scratch_types=[
          pltpu.VMEM((CH, CK), jnp.int32),
          pltpu.VMEM((CK, 16), jnp.float32),
          pltpu.VMEM_SHARED((ZR, 16), jnp.float32),
      ],
  )
  def deg_k(dst_hbm, zero_hbm, ones_hbm, out_hbm, dstv, onesv, dsh):
    c = lax.axis_index("c")
    s = lax.axis_index("s")
    wid = s * NCORE + c
    base = s * RPT
    pltpu.sync_copy(zero_hbm.at[pl.ds(base, RPT)], dsh.at[pl.ds(base, RPT)])
    pltpu.sync_copy(dst_hbm.at[wid], dstv)
    pltpu.sync_copy(ones_hbm, onesv)
    plsc.subcore_barrier()

    @pl.loop(0, CH)
    def _(j):
      pltpu.sync_copy(onesv, dsh.at[dstv.at[j]], add=True)

    plsc.subcore_barrier()
    pltpu.sync_copy(dsh.at[pl.ds(base, RPT)], out_hbm.at[c, pl.ds(base, RPT)])

  return deg_k(dstp, zeros16, ones_rows)


def _agg(eidx, ya, yb, d):
  hd = d // 2
  NB = 4 if hd <= 64 else 2
  NIB = 8
  NG = CH2

  @functools.partial(
      pl.kernel,
      out_type=jax.ShapeDtypeStruct((ZR, d), jnp.float32),
      mesh=_mesh,
      compiler_params=_sc_params,
      scratch_types=(
          [pltpu.VMEM((2, CK), jnp.int32)] * NIB
          + [pltpu.VMEM((CK, hd), jnp.float32)] * NB
          + [pltpu.VMEM_SHARED((ZR, hd), jnp.float32)]
          + [pltpu.VMEM_SHARED((ZR, hd), jnp.float32)]
          + [pltpu.SemaphoreType.DMA] * (NIB + 2 * NB)
      ),
  )
  def agg_k(eidx_hbm, ya_hbm, yb_hbm, out_hbm, *scr):
    dis = scr[:NIB]
    bufs = scr[NIB:NIB + NB]
    ysp, zsh = scr[NIB + NB], scr[NIB + NB + 1]
    isems = scr[NIB + NB + 2:2 * NIB + NB + 2]
    gsems = scr[2 * NIB + NB + 2:2 * NIB + 2 * NB + 2]
    ssems = scr[2 * NIB + 2 * NB + 2:]
    c = lax.axis_index("c")
    s = lax.axis_index("s")
    base = s * RPT

    @pl.when(c == 0)
    def _():
      pltpu.sync_copy(ya_hbm.at[pl.ds(base, RPT)], zsh.at[pl.ds(base, RPT)])
      pltpu.sync_copy(ya_hbm.at[pl.ds(base, RPT)], ysp.at[pl.ds(base, RPT)])

    @pl.when(c == 1)
    def _():
      pltpu.sync_copy(yb_hbm.at[pl.ds(base, RPT)], zsh.at[pl.ds(base, RPT)])
      pltpu.sync_copy(yb_hbm.at[pl.ds(base, RPT)], ysp.at[pl.ds(base, RPT)])

    plsc.subcore_barrier()

    def load_idx(g, islot):
      pltpu.async_copy(eidx_hbm.at[s, g], dis[islot], isems[islot])

    def wait_idx(g, islot):
      pltpu.make_async_copy(eidx_hbm.at[s, g], dis[islot],
                            isems[islot]).wait()

    def gather(g, bslot, islot):
      pltpu.async_copy(ysp.at[dis[islot].at[0]], bufs[bslot], gsems[bslot])

    def drain_gather(g, bslot, islot):
      pltpu.make_async_copy(ysp.at[dis[islot].at[0]], bufs[bslot],
                            gsems[bslot]).wait()

    def scatter(bslot, islot):
      pltpu.async_copy(bufs[bslot], zsh.at[dis[islot].at[1]], ssems[bslot],
                       add=True)

    def drain_scatter(bslot, islot):
      pltpu.make_async_copy(bufs[bslot], zsh.at[dis[islot].at[1]],
                            ssems[bslot]).wait()

    for q in range(NIB - 1):
      load_idx(q, q)
    for q in range(NB - 1):
      wait_idx(q, q)
      gather(q, q, q)

    @pl.loop(0, NG // NIB)
    def _(t):
      for u in range(NIB):
        g = t * NIB + u
        bslot = u % NB
        fslot = (u + NB - 1) % NB
        filot = (u + NIB - 1) % NIB

        @pl.when(g >= 1)
        def _():
          drain_scatter(fslot, filot)

        @pl.when(g + NIB - 1 < NG)
        def _():
          load_idx(g + NIB - 1, filot)

        @pl.when(g + NB - 1 < NG)
        def _():
          wait_idx(g + NB - 1, (u + NB - 1) % NIB)
          gather(g + NB - 1, fslot, (u + NB - 1) % NIB)

        drain_gather(g, bslot, u)
        scatter(bslot, u)

    drain_scatter((NG - 1) % NB, (NG - 1) % NIB)

    plsc.subcore_barrier()
    pltpu.sync_copy(zsh.at[pl.ds(base, RPT)],
                    out_hbm.at[pl.ds(base, RPT), pl.ds(c * hd, hd)])

  return agg_k(eidx, ya, yb)


def _tc_prep(degp, nodes_p):
  def body(deg_ref, x_ref, dinv_ref, ya_ref, yb_ref):
    deg = deg_ref[0, :, 0:1] + deg_ref[1, :, 0:1] + 1.0
    dinv = lax.rsqrt(deg)
    dinv_ref[...] = dinv
    y = x_ref[...] * dinv
    ya_ref[...] = y[:, :D // 2]
    yb_ref[...] = y[:, D // 2:]

  return pl.pallas_call(
      body,
      out_shape=(
          jax.ShapeDtypeStruct((ZR, 1), jnp.float32),
          jax.ShapeDtypeStruct((ZR, D // 2), jnp.float32),
          jax.ShapeDtypeStruct((ZR, D // 2), jnp.float32),
      ),
  )(degp, nodes_p)


def _tc_mid(z1, dinv, W1, b1, W2p):
  def body(z_ref, dinv_ref, w1_ref, b1_ref, w2_ref, ya_ref, yb_ref):
    dinv = dinv_ref[...]
    zs = z_ref[...] * dinv
    h = jnp.dot(zs, w1_ref[...], preferred_element_type=jnp.float32)
    h = jnp.maximum(h + b1_ref[...], 0.0)
    y2 = jnp.dot(h, w2_ref[...], preferred_element_type=jnp.float32)
    y2 = y2 * dinv
    ya_ref[...] = y2[:, :80]
    yb_ref[...] = y2[:, 80:]

  return pl.pallas_call(
      body,
      out_shape=(
          jax.ShapeDtypeStruct((ZR, 80), jnp.float32),
          jax.ShapeDtypeStruct((ZR, 80), jnp.float32),
      ),
  )(z1, dinv, W1, b1.reshape(1, -1), W2p)


def _tc_tail(z2, dinv, b2p, W3p, b3, W4, b4):
  def body(z_ref, dinv_ref, b2_ref, w3_ref, b3_ref, w4_ref, b4_ref, o_ref):
    dinv = dinv_ref[...]
    h2 = jnp.maximum(z_ref[...] * dinv + b2_ref[...], 0.0)
    h3 = jnp.dot(h2, w3_ref[...], preferred_element_type=jnp.float32)
    h3 = jnp.maximum(h3 + b3_ref[...], 0.0)
    o = jnp.dot(h3, w4_ref[...], preferred_element_type=jnp.float32)
    o_ref[...] = jax.nn.sigmoid(o + b4_ref[...])

  return pl.pallas_call(
      body,
      out_shape=jax.ShapeDtypeStruct((ZR, 1), jnp.float32),
  )(z2, dinv, b2p.reshape(1, -1), W3p, b3.reshape(1, -1), W4, b4.reshape(1, -1))


def kernel(nodes, edges, W1, b1, W2, b2, W3, b3, W4, b4):
  f32 = jnp.float32
  nodes_p = jnp.zeros((ZR, D), f32).at[:N].set(nodes)
  pad = EP - E
  src_flat = jnp.concatenate([edges[0], jnp.zeros((pad,), jnp.int32)])
  dst_flat = jnp.concatenate([edges[1], jnp.full((pad,), N, jnp.int32)])

  degp = _deg_kernel(dst_flat.reshape(NW, CH, CK),
                     jnp.zeros((ZR, 16), f32), jnp.ones((CK, 16), f32))
  dinv, y1a, y1b = _tc_prep(degp, nodes_p)

  eidx = jnp.stack([src_flat.reshape(NSUB, CH2, CK),
                    dst_flat.reshape(NSUB, CH2, CK)], axis=2)
  z1 = _agg(eidx, y1a, y1b, D)

  W2p = jnp.zeros((200, 160), f32).at[:, :150].set(W2)
  y2a, y2b = _tc_mid(z1, dinv, W1, b1, W2p)
  z2 = _agg(eidx, y2a, y2b, 160)

  b2p = jnp.zeros((160,), f32).at[:150].set(b2)
  W3p = jnp.zeros((160, 100), f32).at[:150].set(W3)
  out = _tc_tail(z2, dinv, b2p, W3p, b3, W4, b4)
  return out[:N]
